# baseline (speedup 1.0000x reference)
import functools

import jax
import jax.numpy as jnp
from jax import lax
from jax.experimental import pallas as pl
from jax.experimental.pallas import tpu as pltpu
from jax.experimental.pallas import tpu_sc as plsc

BATCH = 16384
FACTOR = 32
VOCAB = 1000000
ROWS128 = VOCAB * FACTOR // 128

_info = plsc.get_sparse_core_info()
_NC, _NS, _L = _info.num_cores, _info.num_subcores, _info.num_lanes
_NW = _NC * _NS
_BPW = BATCH // _NW
_CH = 256
_NCH = _BPW // _CH

_mesh = plsc.VectorSubcoreMesh(core_axis_name="c", subcore_axis_name="s")


@functools.partial(
    pl.kernel,
    mesh=_mesh,
    compiler_params=pltpu.CompilerParams(needs_layout_passes=False),
    out_type=(
        jax.ShapeDtypeStruct((BATCH,), jnp.float32),
        jax.ShapeDtypeStruct((BATCH,), jnp.float32),
    ),
    scratch_types=[
        pltpu.VMEM((_BPW,), jnp.int32),
        pltpu.VMEM((_BPW,), jnp.int32),
        pltpu.VMEM((_BPW,), jnp.int32),
        pltpu.VMEM((_CH,), jnp.int32),
        pltpu.VMEM((_CH,), jnp.int32),
        pltpu.VMEM((_CH,), jnp.int32),
        pltpu.VMEM((_CH, 128), jnp.float32),
        pltpu.VMEM((_CH, 128), jnp.float32),
        pltpu.VMEM((_CH, 128), jnp.float32),
        pltpu.VMEM((_BPW,), jnp.float32),
        pltpu.VMEM((_BPW,), jnp.float32),
        pltpu.SemaphoreType.DMA,
    ],
)
def _bpr_sc(user_hbm, item_i_hbm, item_j_hbm, uw_hbm, iw_hbm,
            out_i_hbm, out_j_hbm,
            u_idx, i_idx, j_idx, u_bid, i_bid, j_bid,
            u_buf, i_buf, j_buf, pred_i_v, pred_j_v, sem):
    wid = lax.axis_index("s") * _NC + lax.axis_index("c")
    base = wid * _BPW

    pltpu.sync_copy(user_hbm.at[pl.ds(base, _BPW)], u_idx)
    pltpu.sync_copy(item_i_hbm.at[pl.ds(base, _BPW)], i_idx)
    pltpu.sync_copy(item_j_hbm.at[pl.ds(base, _BPW)], j_idx)

    for c in range(_NCH):
        off = c * _CH

        def bid_body(g, carry):
            u0 = g * _L
            for idx_ref, bid_ref in ((u_idx, u_bid), (i_idx, i_bid),
                                     (j_idx, j_bid)):
                bid_ref[pl.ds(u0, _L)] = idx_ref[pl.ds(off + u0, _L)] >> 2
            return carry

        lax.fori_loop(0, _CH // _L, bid_body, 0)

        cp_u = pltpu.async_copy(uw_hbm.at[u_bid], u_buf, sem)
        cp_i = pltpu.async_copy(iw_hbm.at[i_bid], i_buf, sem)
        cp_j = pltpu.async_copy(iw_hbm.at[j_bid], j_buf, sem)
        cp_u.wait()
        cp_i.wait()
        cp_j.wait()

        def dot_body(g, carry):
            u0 = g * _L
            rows = u0 + lax.iota(jnp.int32, _L)
            lane_u = (u_idx[pl.ds(off + u0, _L)] & 3) * 32
            lane_i = (i_idx[pl.ds(off + u0, _L)] & 3) * 32
            lane_j = (j_idx[pl.ds(off + u0, _L)] & 3) * 32
            acc_i = jnp.zeros((_L,), jnp.float32)
            acc_j = jnp.zeros((_L,), jnp.float32)
            for f in range(FACTOR):
                uv = plsc.load_gather(u_buf, [rows, lane_u + f])
                iv = plsc.load_gather(i_buf, [rows, lane_i + f])
                jv = plsc.load_gather(j_buf, [rows, lane_j + f])
                acc_i = acc_i + uv * iv
                acc_j = acc_j + uv * jv
            pred_i_v[pl.ds(off + u0, _L)] = acc_i
            pred_j_v[pl.ds(off + u0, _L)] = acc_j
            return carry

        lax.fori_loop(0, _CH // _L, dot_body, 0)

    pltpu.sync_copy(pred_i_v, out_i_hbm.at[pl.ds(base, _BPW)])
    pltpu.sync_copy(pred_j_v, out_j_hbm.at[pl.ds(base, _BPW)])


def kernel(user, item_i, item_j, embed_user_weight, embed_item_weight):
    user = user.astype(jnp.int32)
    item_i = item_i.astype(jnp.int32)
    item_j = item_j.astype(jnp.int32)
    uw = embed_user_weight.reshape(ROWS128, 128)
    iw = embed_item_weight.reshape(ROWS128, 128)
    return _bpr_sc(user, item_i, item_j, uw, iw)
